# baseline (speedup 1.0000x reference)
import functools

import jax
import jax.numpy as jnp
from jax import lax
from jax.experimental import pallas as pl
from jax.experimental.pallas import tpu as pltpu
from jax.experimental.pallas import tpu_sc as plsc

_DIM1 = 120000
_N = 256
_BLOCK = 2400
_GRID = _DIM1 // _BLOCK
_ROWS = (3, 10, 12089)

_NC = 2
_NS = 16
_NW = 30
_WROWS = _DIM1 // _NW
_CHUNK = 400
_NCHUNK = _WROWS // _CHUNK


def _tc_body(row0_ref, out_ref):
    i = pl.program_id(0)
    ids = jax.lax.broadcasted_iota(jnp.int32, (_BLOCK, 1), 0) + i * _BLOCK
    mask = (ids == _ROWS[0]) | (ids == _ROWS[1]) | (ids == _ROWS[2])
    out_ref[...] = jnp.where(mask, row0_ref[...], 0.0)


def _sc_fill(row0_hbm, out_hbm, zbuf, rowv, sem):
    c = lax.axis_index("c")
    s = lax.axis_index("s")
    wid = s * _NC + c
    zeros16 = jnp.zeros((16,), jnp.float32)

    def _zero_row(r, carry):
        for j in range(_N // 16):
            zbuf[r, pl.ds(j * 16, 16)] = zeros16
        return carry

    lax.fori_loop(0, _CHUNK, _zero_row, 0)

    @pl.when(wid < _NW)
    def _():
        base = wid * _WROWS
        copies = [
            pltpu.async_copy(zbuf, out_hbm.at[pl.ds(base + k * _CHUNK, _CHUNK)], sem)
            for k in range(_NCHUNK)
        ]
        for cp in copies:
            cp.wait()

    @pl.when(wid == 0)
    def _():
        pltpu.sync_copy(row0_hbm, rowv)
        for j in range(_N // 16):
            v = rowv[0, pl.ds(j * 16, 16)]
            zbuf[_ROWS[0], pl.ds(j * 16, 16)] = v
            zbuf[_ROWS[1], pl.ds(j * 16, 16)] = v
        pltpu.sync_copy(zbuf.at[pl.ds(0, 16)], out_hbm.at[pl.ds(0, 16)])

    @pl.when(wid == _ROWS[2] // _WROWS)
    def _():
        pltpu.sync_copy(row0_hbm, rowv)
        for j in range(_N // 16):
            zbuf[_ROWS[2] % 8, pl.ds(j * 16, 16)] = rowv[0, pl.ds(j * 16, 16)]
        pltpu.sync_copy(
            zbuf.at[pl.ds(0, 8)], out_hbm.at[pl.ds(_ROWS[2] - _ROWS[2] % 8, 8)]
        )


def kernel(arr2):
    row0 = arr2[0:1, :]
    out1 = pl.pallas_call(
        _tc_body,
        grid=(_GRID,),
        in_specs=[pl.BlockSpec((1, _N), lambda i: (0, 0))],
        out_specs=pl.BlockSpec((_BLOCK, _N), lambda i: (i, 0)),
        out_shape=jax.ShapeDtypeStruct((_DIM1, _N), jnp.float32),
    )(row0)
    sc_fill = functools.partial(
        pl.kernel,
        mesh=plsc.VectorSubcoreMesh(core_axis_name="c", subcore_axis_name="s"),
        out_type=jax.ShapeDtypeStruct((_DIM1, _N), jnp.float32),
        scratch_types=[
            pltpu.VMEM((_CHUNK, _N), jnp.float32),
            pltpu.VMEM((1, _N), jnp.float32),
            pltpu.SemaphoreType.DMA,
        ],
    )(_sc_fill)
    out2 = sc_fill(row0)
    return (out1, out2)
